# Initial kernel scaffold; baseline (speedup 1.0000x reference)
import jax
import jax.numpy as jnp
from jax.experimental import pallas as pl


def kernel(queries_0, queries_1, keys_0, keys_1, values_0, values_1, topk_pos, weight):
    raise NotImplementedError("write your pallas kernel here")


# speedup vs baseline: 4.2920x; 4.2920x over previous
import functools

import jax
import jax.numpy as jnp
from jax import lax
from jax.experimental import pallas as pl
from jax.experimental.pallas import tpu as pltpu
from jax.experimental.pallas import tpu_sc as plsc

NH = 8
D = 32
P = 16
S = 4
N_TILE = 8
CHUNK = 128
ROW = S * D


def _make_sc_gather(B):
    info = plsc.get_sparse_core_info()
    NW = info.num_cores * info.num_subcores
    assert B % (NW * CHUNK) == 0
    b_per_w = B // NW
    n_chunks = b_per_w // CHUNK
    mesh = plsc.VectorSubcoreMesh(core_axis_name="c", subcore_axis_name="s")
    out_sds = jax.ShapeDtypeStruct((B, ROW), jnp.float32)

    @functools.partial(
        pl.kernel,
        mesh=mesh,
        out_type=(out_sds, out_sds),
        scratch_types=[
            pltpu.VMEM((CHUNK,), jnp.int32),
            pltpu.VMEM((CHUNK, ROW), jnp.float32),
            pltpu.VMEM((CHUNK, ROW), jnp.float32),
            pltpu.SemaphoreType.DMA,
            pltpu.SemaphoreType.DMA,
        ],
    )
    def gather2(ktab, vtab, idx_hbm, outk, outv, idx_v, krows, vrows, sem1, sem2):
        wid = lax.axis_index("s") * info.num_cores + lax.axis_index("c")
        base = wid * b_per_w

        def body(i, carry):
            off = base + i * CHUNK
            pltpu.sync_copy(idx_hbm.at[pl.ds(off, CHUNK)], idx_v)
            ck = pltpu.async_copy(ktab.at[idx_v], krows, sem1)
            cv = pltpu.async_copy(vtab.at[idx_v], vrows, sem2)
            ck.wait()
            cv.wait()
            pltpu.sync_copy(krows, outk.at[pl.ds(off, CHUNK)])
            pltpu.sync_copy(vrows, outv.at[pl.ds(off, CHUNK)])
            return carry

        lax.fori_loop(0, n_chunks, body, 0)

    return gather2


def _softmax_ps(qk):
    mx = jnp.max(jnp.max(qk, axis=4, keepdims=True), axis=2, keepdims=True)
    e = jnp.exp(qk - mx)
    den = jnp.sum(jnp.sum(e, axis=4, keepdims=True), axis=2, keepdims=True)
    return e / den


def _trunc(x):
    return x.astype(jnp.bfloat16).astype(jnp.float32)


def _qk_all(q_ref, kg_ref):
    scale = 1.0 / D ** 0.5
    kg = _trunc(kg_ref[...])
    qs = _trunc(q_ref[...])
    qks = []
    for t in range(4):
        qt = qs[:, t][:, None, :, None, :]
        qks.append(jnp.sum(qt * kg, axis=-1) * scale)
    return jnp.stack(qks, axis=1)


def _message(a, vg_ref, msg_ref):
    vg = _trunc(vg_ref[...])
    for t in range(4):
        w = _trunc(a[:, t])[..., None] * vg
        msg_ref[:, t] = jnp.sum(jnp.sum(w, axis=3), axis=1)


def _attn_body_l0(q_ref, kg_ref, vg_ref, idx_ref, msg_ref, tki_ref):
    qk = _qk_all(q_ref, kg_ref)
    _message(_softmax_ps(qk), vg_ref, msg_ref)
    idxv = idx_ref[...][:, None]
    kiota = (S * lax.broadcasted_iota(jnp.int32, qk.shape, 2)
             + lax.broadcasted_iota(jnp.int32, qk.shape, 4))
    work = qk
    for j in range(16):
        m = jnp.max(jnp.max(work, axis=4, keepdims=True), axis=2, keepdims=True)
        cand = jnp.where(work == m, kiota, P * S)
        first = jnp.min(jnp.min(cand, axis=4, keepdims=True), axis=2, keepdims=True)
        onehot = kiota == first
        pick = jnp.sum(jnp.sum(jnp.where(onehot, idxv, 0), axis=4), axis=2)
        tki_ref[:, :, j, :] = pick
        work = jnp.where(onehot, -1e30, work)


def _attn_body_l1(q_ref, kg_ref, vg_ref, m0_ref, w1_ref, out_ref):
    qk = _qk_all(q_ref, kg_ref)
    a = _softmax_ps(qk)
    w1 = w1_ref[0, 0]
    vg = _trunc(vg_ref[...])
    m0 = m0_ref[...]
    for t in range(4):
        w = _trunc(a[:, t])[..., None] * vg
        out_ref[:, t] = m0 + jnp.sum(jnp.sum(w, axis=3), axis=1) * w1


def _run_l0(q, kg, vg, idxv):
    M = q.shape[0]
    bs4 = pl.BlockSpec((N_TILE, 4, NH, D), lambda i: (i, 0, 0, 0))
    bsk = pl.BlockSpec((N_TILE, P, NH, S, D), lambda i: (i, 0, 0, 0, 0))
    bsi = pl.BlockSpec((N_TILE, P, NH, S), lambda i: (i, 0, 0, 0))
    bst = pl.BlockSpec((N_TILE, 4, 16, NH), lambda i: (i, 0, 0, 0))
    return pl.pallas_call(
        _attn_body_l0,
        grid=(M // N_TILE,),
        in_specs=[bs4, bsk, bsk, bsi],
        out_specs=[bs4, bst],
        out_shape=[
            jax.ShapeDtypeStruct((M, 4, NH, D), jnp.float32),
            jax.ShapeDtypeStruct((M, 4, 16, NH), jnp.int32),
        ],
    )(q, kg, vg, idxv)


def _run_l1(q, kg, vg, m0, w1):
    M = q.shape[0]
    bs4 = pl.BlockSpec((N_TILE, 4, NH, D), lambda i: (i, 0, 0, 0))
    bsk = pl.BlockSpec((N_TILE, P, NH, S, D), lambda i: (i, 0, 0, 0, 0))
    bsm = pl.BlockSpec((N_TILE, NH, D), lambda i: (i, 0, 0))
    bsw = pl.BlockSpec(memory_space=pltpu.SMEM)
    return pl.pallas_call(
        _attn_body_l1,
        grid=(M // N_TILE,),
        in_specs=[bs4, bsk, bsk, bsm, bsw],
        out_specs=bs4,
        out_shape=jax.ShapeDtypeStruct((M, 4, NH, D), jnp.float32),
    )(q, kg, vg, m0, w1)


def _quad_tables(key, value, bs, side):
    def go(t):
        t = jnp.transpose(t, (0, 2, 3, 1))
        t = t.reshape(bs, side // 2, 2, side // 2, 2, NH, D)
        t = jnp.transpose(t, (0, 1, 3, 5, 2, 4, 6))
        return t.reshape(bs * (side // 2) ** 2 * NH, ROW)
    return go(key), go(value)


def _quad_queries(q, bs, side):
    nq = (side // 2) ** 2
    q = q.reshape(bs, NH * D, side // 2, 2, side // 2, 2)
    q = jnp.transpose(q, (0, 2, 4, 3, 5, 1))
    return q.reshape(bs * nq, 4, NH, D)


def _gather_level(keym, valm, quad_idx, bs, side):
    nq = quad_idx.shape[1]
    ktab, vtab = _quad_tables(keym, valm, bs, side)
    nquad = (side // 2) ** 2
    g = (jnp.arange(bs, dtype=jnp.int32)[:, None, None, None] * nquad
         + quad_idx) * NH + jnp.arange(NH, dtype=jnp.int32)
    B = bs * nq * P * NH
    kg, vg = _make_sc_gather(B)(ktab, vtab, g.reshape(-1))
    shape = (bs * nq, P, NH, S, D)
    return kg.reshape(shape), vg.reshape(shape)


def kernel(queries_0, queries_1, keys_0, keys_1, values_0, values_1, topk_pos, weight):
    bs = queries_1.shape[0]

    tp = topk_pos.astype(jnp.int32)
    r, c = tp[0], tp[1]
    q0 = _quad_queries(queries_1, bs, 32)
    kg0, vg0 = _gather_level(keys_1, values_1, r * 16 + c, bs, 32)
    sub_r = jnp.array([0, 0, 1, 1], jnp.int32)
    sub_c = jnp.array([0, 1, 0, 1], jnp.int32)
    idxv0 = ((2 * r[..., None] + sub_r) * 32
             + 2 * c[..., None] + sub_c)
    idxv0 = idxv0.reshape(bs * 256, P, NH, S)
    msg0, tki0 = _run_l0(q0, kg0, vg0, idxv0)

    tki = tki0.reshape(bs, 16, 16, 2, 2, 16, NH)
    tki = jnp.transpose(tki, (0, 1, 3, 2, 4, 5, 6)).reshape(bs, 1024, 16, NH)

    q1 = _quad_queries(queries_0, bs, 64)
    kg1, vg1 = _gather_level(keys_0, values_0, tki, bs, 64)

    ws = jax.nn.softmax(weight, axis=0)
    m0 = msg0.reshape(bs, 256, 4, NH, D) * ws[0]
    m0 = m0.reshape(bs, 64, 4, 2, 2, NH, D)
    m0 = jnp.transpose(m0, (0, 1, 3, 2, 4, 5, 6)).reshape(bs * 1024, NH, D)
    fin = _run_l1(q1, kg1, vg1, m0, ws[1].reshape(1, 1))

    fin = fin.reshape(bs, 32, 32, 2, 2, NH, D)
    fin = jnp.transpose(fin, (0, 1, 3, 2, 4, 5, 6)).reshape(bs, 4096, NH, D)
    return fin


# speedup vs baseline: 9.7747x; 2.2774x over previous
import functools

import jax
import jax.numpy as jnp
from jax import lax
from jax.experimental import pallas as pl
from jax.experimental.pallas import tpu as pltpu
from jax.experimental.pallas import tpu_sc as plsc

NH = 8
D = 32
P = 16
S = 4
N_TILE = 8
CHUNK = 128
ROW = S * D


def _make_sc_gather(B):
    info = plsc.get_sparse_core_info()
    NW = info.num_cores * info.num_subcores
    assert B % (NW * CHUNK) == 0
    b_per_w = B // NW
    n_chunks = b_per_w // CHUNK
    mesh = plsc.VectorSubcoreMesh(core_axis_name="c", subcore_axis_name="s")
    out_sds = jax.ShapeDtypeStruct((B, ROW), jnp.float32)

    @functools.partial(
        pl.kernel,
        mesh=mesh,
        out_type=(out_sds, out_sds),
        scratch_types=[
            pltpu.VMEM((CHUNK,), jnp.int32),
            pltpu.VMEM((CHUNK, ROW), jnp.float32),
            pltpu.VMEM((CHUNK, ROW), jnp.float32),
            pltpu.SemaphoreType.DMA,
            pltpu.SemaphoreType.DMA,
        ],
    )
    def gather2(ktab, vtab, idx_hbm, outk, outv, idx_v, krows, vrows, sem1, sem2):
        wid = lax.axis_index("s") * info.num_cores + lax.axis_index("c")
        base = wid * b_per_w

        def body(i, carry):
            off = base + i * CHUNK
            pltpu.sync_copy(idx_hbm.at[pl.ds(off, CHUNK)], idx_v)
            ck = pltpu.async_copy(ktab.at[idx_v], krows, sem1)
            cv = pltpu.async_copy(vtab.at[idx_v], vrows, sem2)
            ck.wait()
            cv.wait()
            pltpu.sync_copy(krows, outk.at[pl.ds(off, CHUNK)])
            pltpu.sync_copy(vrows, outv.at[pl.ds(off, CHUNK)])
            return carry

        lax.fori_loop(0, n_chunks, body, 0)

    return gather2


def _softmax_ps(qk):
    mx = jnp.max(jnp.max(qk, axis=4, keepdims=True), axis=2, keepdims=True)
    e = jnp.exp(qk - mx)
    den = jnp.sum(jnp.sum(e, axis=4, keepdims=True), axis=2, keepdims=True)
    return e / den


def _trunc(x):
    return x.astype(jnp.bfloat16).astype(jnp.float32)


def _seg():
    return (lax.broadcasted_iota(jnp.int32, (ROW, S), 0) // D
            == lax.broadcasted_iota(jnp.int32, (ROW, S), 1)).astype(jnp.float32)


def _qk_all(q_ref, kg_ref):
    n = q_ref.shape[0]
    scale = 1.0 / D ** 0.5
    kg = _trunc(kg_ref[...])
    qs = _trunc(q_ref[...])
    seg = _seg()
    qks = []
    for t in range(4):
        qt = qs[:, t]
        qb = jnp.concatenate([qt] * S, axis=-1)[:, None]
        prod = (kg * qb).reshape(n * P * NH, ROW)
        qk = jax.lax.dot(prod, seg, precision=jax.lax.Precision.HIGHEST)
        qks.append(qk.reshape(n, P, NH, S) * scale)
    return jnp.stack(qks, axis=1)


def _message_t(a_t, vg, seg):
    n = vg.shape[0]
    ab = jax.lax.dot(_trunc(a_t).reshape(n * P * NH, S), seg.T,
                     precision=jax.lax.Precision.HIGHEST)
    w = jnp.sum(ab.reshape(n, P, NH, ROW) * vg, axis=1)
    return sum(w[..., s * D:(s + 1) * D] for s in range(S))


def _attn_body_l0(q_ref, kg_ref, vg_ref, idx_ref, msg_ref, tki_ref):
    qk = _qk_all(q_ref, kg_ref)
    a = _softmax_ps(qk)
    vg = _trunc(vg_ref[...])
    seg = _seg()
    for t in range(4):
        msg_ref[:, t] = _message_t(a[:, t], vg, seg)
    idxv = idx_ref[...][:, None]
    kiota = (S * lax.broadcasted_iota(jnp.int32, qk.shape, 2)
             + lax.broadcasted_iota(jnp.int32, qk.shape, 4))
    work = qk
    for j in range(16):
        m = jnp.max(jnp.max(work, axis=4, keepdims=True), axis=2, keepdims=True)
        cand = jnp.where(work == m, kiota, P * S)
        first = jnp.min(jnp.min(cand, axis=4, keepdims=True), axis=2, keepdims=True)
        onehot = kiota == first
        pick = jnp.sum(jnp.sum(jnp.where(onehot, idxv, 0), axis=4), axis=2)
        tki_ref[:, :, j, :] = pick
        work = jnp.where(onehot, -1e30, work)


def _attn_body_l1(q_ref, kg_ref, vg_ref, m0_ref, w1_ref, out_ref):
    qk = _qk_all(q_ref, kg_ref)
    a = _softmax_ps(qk)
    w1 = w1_ref[0, 0]
    vg = _trunc(vg_ref[...])
    m0 = m0_ref[...]
    seg = _seg()
    for t in range(4):
        out_ref[:, t] = m0 + _message_t(a[:, t], vg, seg) * w1


def _run_l0(q, kg, vg, idxv):
    M = q.shape[0]
    bs4 = pl.BlockSpec((N_TILE, 4, NH, D), lambda i: (i, 0, 0, 0))
    bsk = pl.BlockSpec((N_TILE, P, NH, ROW), lambda i: (i, 0, 0, 0))
    bsi = pl.BlockSpec((N_TILE, P, NH, S), lambda i: (i, 0, 0, 0))
    bst = pl.BlockSpec((N_TILE, 4, 16, NH), lambda i: (i, 0, 0, 0))
    return pl.pallas_call(
        _attn_body_l0,
        grid=(M // N_TILE,),
        in_specs=[bs4, bsk, bsk, bsi],
        out_specs=[bs4, bst],
        out_shape=[
            jax.ShapeDtypeStruct((M, 4, NH, D), jnp.float32),
            jax.ShapeDtypeStruct((M, 4, 16, NH), jnp.int32),
        ],
    )(q, kg, vg, idxv)


def _run_l1(q, kg, vg, m0, w1):
    M = q.shape[0]
    bs4 = pl.BlockSpec((N_TILE, 4, NH, D), lambda i: (i, 0, 0, 0))
    bsk = pl.BlockSpec((N_TILE, P, NH, ROW), lambda i: (i, 0, 0, 0))
    bsm = pl.BlockSpec((N_TILE, NH, D), lambda i: (i, 0, 0))
    bsw = pl.BlockSpec(memory_space=pltpu.SMEM)
    return pl.pallas_call(
        _attn_body_l1,
        grid=(M // N_TILE,),
        in_specs=[bs4, bsk, bsk, bsm, bsw],
        out_specs=bs4,
        out_shape=jax.ShapeDtypeStruct((M, 4, NH, D), jnp.float32),
    )(q, kg, vg, m0, w1)


def _quad_tables(key, value, bs, side):
    def go(t):
        t = jnp.transpose(t, (0, 2, 3, 1))
        t = t.reshape(bs, side // 2, 2, side // 2, 2, NH, D)
        t = jnp.transpose(t, (0, 1, 3, 5, 2, 4, 6))
        return t.reshape(bs * (side // 2) ** 2 * NH, ROW)
    return go(key), go(value)


def _quad_queries(q, bs, side):
    nq = (side // 2) ** 2
    q = q.reshape(bs, NH * D, side // 2, 2, side // 2, 2)
    q = jnp.transpose(q, (0, 2, 4, 3, 5, 1))
    return q.reshape(bs * nq, 4, NH, D)


def _gather_level(keym, valm, quad_idx, bs, side):
    nq = quad_idx.shape[1]
    ktab, vtab = _quad_tables(keym, valm, bs, side)
    nquad = (side // 2) ** 2
    g = (jnp.arange(bs, dtype=jnp.int32)[:, None, None, None] * nquad
         + quad_idx) * NH + jnp.arange(NH, dtype=jnp.int32)
    B = bs * nq * P * NH
    kg, vg = _make_sc_gather(B)(ktab, vtab, g.reshape(-1))
    shape = (bs * nq, P, NH, ROW)
    return kg.reshape(shape), vg.reshape(shape)


def kernel(queries_0, queries_1, keys_0, keys_1, values_0, values_1, topk_pos, weight):
    bs = queries_1.shape[0]

    tp = topk_pos.astype(jnp.int32)
    r, c = tp[0], tp[1]
    q0 = _quad_queries(queries_1, bs, 32)
    kg0, vg0 = _gather_level(keys_1, values_1, r * 16 + c, bs, 32)
    sub_r = jnp.array([0, 0, 1, 1], jnp.int32)
    sub_c = jnp.array([0, 1, 0, 1], jnp.int32)
    idxv0 = ((2 * r[..., None] + sub_r) * 32
             + 2 * c[..., None] + sub_c)
    idxv0 = idxv0.reshape(bs * 256, P, NH, S)
    msg0, tki0 = _run_l0(q0, kg0, vg0, idxv0)

    tki = tki0.reshape(bs, 16, 16, 2, 2, 16, NH)
    tki = jnp.transpose(tki, (0, 1, 3, 2, 4, 5, 6)).reshape(bs, 1024, 16, NH)

    q1 = _quad_queries(queries_0, bs, 64)
    kg1, vg1 = _gather_level(keys_0, values_0, tki, bs, 64)

    ws = jax.nn.softmax(weight, axis=0)
    m0 = msg0.reshape(bs, 256, 4, NH, D) * ws[0]
    m0 = m0.reshape(bs, 64, 4, 2, 2, NH, D)
    m0 = jnp.transpose(m0, (0, 1, 3, 2, 4, 5, 6)).reshape(bs * 1024, NH, D)
    fin = _run_l1(q1, kg1, vg1, m0, ws[1].reshape(1, 1))

    fin = fin.reshape(bs, 32, 32, 2, 2, NH, D)
    fin = jnp.transpose(fin, (0, 1, 3, 2, 4, 5, 6)).reshape(bs, 4096, NH, D)
    return fin


# speedup vs baseline: 9.8444x; 1.0071x over previous
import functools

import jax
import jax.numpy as jnp
from jax import lax
from jax.experimental import pallas as pl
from jax.experimental.pallas import tpu as pltpu
from jax.experimental.pallas import tpu_sc as plsc

NH = 8
D = 32
P = 16
S = 4
N_TILE = 8
CHUNK = 128
ROW = S * D


def _make_sc_gather(B):
    info = plsc.get_sparse_core_info()
    NW = info.num_cores * info.num_subcores
    assert B % (NW * CHUNK) == 0
    b_per_w = B // NW
    n_chunks = b_per_w // CHUNK
    mesh = plsc.VectorSubcoreMesh(core_axis_name="c", subcore_axis_name="s")
    out_sds = jax.ShapeDtypeStruct((B, ROW), jnp.float32)

    @functools.partial(
        pl.kernel,
        mesh=mesh,
        out_type=(out_sds, out_sds),
        scratch_types=[
            pltpu.VMEM((CHUNK,), jnp.int32),
            pltpu.VMEM((CHUNK,), jnp.int32),
            pltpu.VMEM((CHUNK, ROW), jnp.float32),
            pltpu.VMEM((CHUNK, ROW), jnp.float32),
            pltpu.VMEM((CHUNK, ROW), jnp.float32),
            pltpu.VMEM((CHUNK, ROW), jnp.float32),
            pltpu.SemaphoreType.DMA,
            pltpu.SemaphoreType.DMA,
            pltpu.SemaphoreType.DMA,
            pltpu.SemaphoreType.DMA,
        ],
    )
    def gather2(ktab, vtab, idx_hbm, outk, outv,
                idx_a, idx_b, krows_a, vrows_a, krows_b, vrows_b,
                semka, semva, semkb, semvb):
        wid = lax.axis_index("s") * info.num_cores + lax.axis_index("c")
        base = wid * b_per_w

        def body(j, carry):
            off_a = base + (2 * j) * CHUNK
            off_b = off_a + CHUNK
            pltpu.sync_copy(idx_hbm.at[pl.ds(off_a, CHUNK)], idx_a)
            cka = pltpu.async_copy(ktab.at[idx_a], krows_a, semka)
            cva = pltpu.async_copy(vtab.at[idx_a], vrows_a, semva)
            pltpu.sync_copy(idx_hbm.at[pl.ds(off_b, CHUNK)], idx_b)
            ckb = pltpu.async_copy(ktab.at[idx_b], krows_b, semkb)
            cvb = pltpu.async_copy(vtab.at[idx_b], vrows_b, semvb)
            cka.wait()
            cva.wait()
            pltpu.sync_copy(krows_a, outk.at[pl.ds(off_a, CHUNK)])
            pltpu.sync_copy(vrows_a, outv.at[pl.ds(off_a, CHUNK)])
            ckb.wait()
            cvb.wait()
            pltpu.sync_copy(krows_b, outk.at[pl.ds(off_b, CHUNK)])
            pltpu.sync_copy(vrows_b, outv.at[pl.ds(off_b, CHUNK)])
            return carry

        lax.fori_loop(0, n_chunks // 2, body, 0)

    return gather2


def _softmax_ps(qk):
    mx = jnp.max(jnp.max(qk, axis=4, keepdims=True), axis=2, keepdims=True)
    e = jnp.exp(qk - mx)
    den = jnp.sum(jnp.sum(e, axis=4, keepdims=True), axis=2, keepdims=True)
    return e / den


def _trunc(x):
    return x.astype(jnp.bfloat16).astype(jnp.float32)


def _seg():
    return (lax.broadcasted_iota(jnp.int32, (ROW, S), 0) // D
            == lax.broadcasted_iota(jnp.int32, (ROW, S), 1)).astype(jnp.float32)


def _qk_all(q_ref, kg_ref):
    n = q_ref.shape[0]
    scale = 1.0 / D ** 0.5
    kg = _trunc(kg_ref[...])
    qs = _trunc(q_ref[...])
    seg = _seg()
    qks = []
    for t in range(4):
        qt = qs[:, t]
        qb = jnp.concatenate([qt] * S, axis=-1)[:, None]
        prod = (kg * qb).reshape(n * P * NH, ROW)
        qk = jax.lax.dot(prod, seg, precision=jax.lax.Precision.HIGHEST)
        qks.append(qk.reshape(n, P, NH, S) * scale)
    return jnp.stack(qks, axis=1)


def _message_t(a_t, vg, seg):
    n = vg.shape[0]
    ab = jax.lax.dot(_trunc(a_t).reshape(n * P * NH, S), seg.T,
                     precision=jax.lax.Precision.HIGHEST)
    w = jnp.sum(ab.reshape(n, P, NH, ROW) * vg, axis=1)
    return sum(w[..., s * D:(s + 1) * D] for s in range(S))


def _attn_body_l0(q_ref, kg_ref, vg_ref, idx_ref, msg_ref, tki_ref):
    qk = _qk_all(q_ref, kg_ref)
    a = _softmax_ps(qk)
    vg = _trunc(vg_ref[...])
    seg = _seg()
    for t in range(4):
        msg_ref[:, t] = _message_t(a[:, t], vg, seg)
    idxv = idx_ref[...][:, None]
    kiota = (S * lax.broadcasted_iota(jnp.int32, qk.shape, 2)
             + lax.broadcasted_iota(jnp.int32, qk.shape, 4))
    work = qk
    for j in range(16):
        m = jnp.max(jnp.max(work, axis=4, keepdims=True), axis=2, keepdims=True)
        cand = jnp.where(work == m, kiota, P * S)
        first = jnp.min(jnp.min(cand, axis=4, keepdims=True), axis=2, keepdims=True)
        onehot = kiota == first
        pick = jnp.sum(jnp.sum(jnp.where(onehot, idxv, 0), axis=4), axis=2)
        tki_ref[:, :, j, :] = pick
        work = jnp.where(onehot, -1e30, work)


def _attn_body_l1(q_ref, kg_ref, vg_ref, m0_ref, w1_ref, out_ref):
    qk = _qk_all(q_ref, kg_ref)
    a = _softmax_ps(qk)
    w1 = w1_ref[0, 0]
    vg = _trunc(vg_ref[...])
    m0 = m0_ref[...]
    seg = _seg()
    for t in range(4):
        out_ref[:, t] = m0 + _message_t(a[:, t], vg, seg) * w1


def _run_l0(q, kg, vg, idxv):
    M = q.shape[0]
    bs4 = pl.BlockSpec((N_TILE, 4, NH, D), lambda i: (i, 0, 0, 0))
    bsk = pl.BlockSpec((N_TILE, P, NH, ROW), lambda i: (i, 0, 0, 0))
    bsi = pl.BlockSpec((N_TILE, P, NH, S), lambda i: (i, 0, 0, 0))
    bst = pl.BlockSpec((N_TILE, 4, 16, NH), lambda i: (i, 0, 0, 0))
    return pl.pallas_call(
        _attn_body_l0,
        grid=(M // N_TILE,),
        in_specs=[bs4, bsk, bsk, bsi],
        out_specs=[bs4, bst],
        out_shape=[
            jax.ShapeDtypeStruct((M, 4, NH, D), jnp.float32),
            jax.ShapeDtypeStruct((M, 4, 16, NH), jnp.int32),
        ],
    )(q, kg, vg, idxv)


def _run_l1(q, kg, vg, m0, w1):
    M = q.shape[0]
    bs4 = pl.BlockSpec((N_TILE, 4, NH, D), lambda i: (i, 0, 0, 0))
    bsk = pl.BlockSpec((N_TILE, P, NH, ROW), lambda i: (i, 0, 0, 0))
    bsm = pl.BlockSpec((N_TILE, NH, D), lambda i: (i, 0, 0))
    bsw = pl.BlockSpec(memory_space=pltpu.SMEM)
    return pl.pallas_call(
        _attn_body_l1,
        grid=(M // N_TILE,),
        in_specs=[bs4, bsk, bsk, bsm, bsw],
        out_specs=bs4,
        out_shape=jax.ShapeDtypeStruct((M, 4, NH, D), jnp.float32),
    )(q, kg, vg, m0, w1)


def _quad_tables(key, value, bs, side):
    def go(t):
        t = jnp.transpose(t, (0, 2, 3, 1))
        t = t.reshape(bs, side // 2, 2, side // 2, 2, NH, D)
        t = jnp.transpose(t, (0, 1, 3, 5, 2, 4, 6))
        return t.reshape(bs * (side // 2) ** 2 * NH, ROW)
    return go(key), go(value)


def _quad_queries(q, bs, side):
    nq = (side // 2) ** 2
    q = q.reshape(bs, NH * D, side // 2, 2, side // 2, 2)
    q = jnp.transpose(q, (0, 2, 4, 3, 5, 1))
    return q.reshape(bs * nq, 4, NH, D)


def _gather_level(keym, valm, quad_idx, bs, side):
    nq = quad_idx.shape[1]
    ktab, vtab = _quad_tables(keym, valm, bs, side)
    nquad = (side // 2) ** 2
    g = (jnp.arange(bs, dtype=jnp.int32)[:, None, None, None] * nquad
         + quad_idx) * NH + jnp.arange(NH, dtype=jnp.int32)
    B = bs * nq * P * NH
    kg, vg = _make_sc_gather(B)(ktab, vtab, g.reshape(-1))
    shape = (bs * nq, P, NH, ROW)
    return kg.reshape(shape), vg.reshape(shape)


def kernel(queries_0, queries_1, keys_0, keys_1, values_0, values_1, topk_pos, weight):
    bs = queries_1.shape[0]

    tp = topk_pos.astype(jnp.int32)
    r, c = tp[0], tp[1]
    q0 = _quad_queries(queries_1, bs, 32)
    kg0, vg0 = _gather_level(keys_1, values_1, r * 16 + c, bs, 32)
    sub_r = jnp.array([0, 0, 1, 1], jnp.int32)
    sub_c = jnp.array([0, 1, 0, 1], jnp.int32)
    idxv0 = ((2 * r[..., None] + sub_r) * 32
             + 2 * c[..., None] + sub_c)
    idxv0 = idxv0.reshape(bs * 256, P, NH, S)
    msg0, tki0 = _run_l0(q0, kg0, vg0, idxv0)

    tki = tki0.reshape(bs, 16, 16, 2, 2, 16, NH)
    tki = jnp.transpose(tki, (0, 1, 3, 2, 4, 5, 6)).reshape(bs, 1024, 16, NH)

    q1 = _quad_queries(queries_0, bs, 64)
    kg1, vg1 = _gather_level(keys_0, values_0, tki, bs, 64)

    ws = jax.nn.softmax(weight, axis=0)
    m0 = msg0.reshape(bs, 256, 4, NH, D) * ws[0]
    m0 = m0.reshape(bs, 64, 4, 2, 2, NH, D)
    m0 = jnp.transpose(m0, (0, 1, 3, 2, 4, 5, 6)).reshape(bs * 1024, NH, D)
    fin = _run_l1(q1, kg1, vg1, m0, ws[1].reshape(1, 1))

    fin = fin.reshape(bs, 32, 32, 2, 2, NH, D)
    fin = jnp.transpose(fin, (0, 1, 3, 2, 4, 5, 6)).reshape(bs, 4096, NH, D)
    return fin
